# baseline (speedup 1.0000x reference)
import functools

import jax
import jax.numpy as jnp
from jax import lax
from jax.experimental import pallas as pl
from jax.experimental.pallas import tpu as pltpu

_S = 30.0


def _half_loss(xb, tgt, ml):
    blk, c = xb.shape
    col = lax.broadcasted_iota(jnp.int32, (blk, c), 1)
    onehot = col == tgt
    m_row = jnp.sum(jnp.where(onehot, ml, 0.0), axis=1, keepdims=True)
    logits = _S * jnp.where(onehot, xb - m_row, xb)
    mx = jnp.max(logits, axis=1, keepdims=True)
    se = jnp.sum(jnp.exp(logits - mx), axis=1, keepdims=True)
    logz = jnp.log(se) + mx
    tgt_logit = jnp.sum(jnp.where(onehot, logits, 0.0), axis=1, keepdims=True)
    return jnp.sum(logz - tgt_logit)


def _ldam_body(x1_ref, x2_ref, tgt1_ref, tgt2_ref, ml_ref, out_ref, *,
               nrows_total):
    ml = ml_ref[...]
    p1 = _half_loss(x1_ref[...], tgt1_ref[...].reshape(-1, 1), ml)
    p2 = _half_loss(x2_ref[...], tgt2_ref[...].reshape(-1, 1), ml)
    part = ((p1 + p2) * (1.0 / nrows_total)).reshape(1, 1)

    @pl.when(pl.program_id(0) == 0)
    def _():
        out_ref[...] = jnp.zeros_like(out_ref)

    out_ref[...] += part


def kernel(x, m_list, target):
    b, c = x.shape
    blk = 2048
    grid = b // blk // 2
    tgt3 = target.reshape(2 * grid, 1, blk)
    out = pl.pallas_call(
        functools.partial(_ldam_body, nrows_total=b),
        grid=(grid,),
        in_specs=[
            pl.BlockSpec((blk, c), lambda i: (i, 0)),
            pl.BlockSpec((blk, c), lambda i: (i + grid, 0)),
            pl.BlockSpec((1, 1, blk), lambda i: (i, 0, 0)),
            pl.BlockSpec((1, 1, blk), lambda i: (i + grid, 0, 0)),
            pl.BlockSpec((1, c), lambda i: (0, 0)),
        ],
        out_specs=pl.BlockSpec((1, 1), lambda i: (0, 0)),
        out_shape=jax.ShapeDtypeStruct((1, 1), jnp.float32),
        compiler_params=pltpu.CompilerParams(
            allow_input_fusion=[True, True, True, True, True]),
    )(x, x, tgt3, tgt3, m_list.reshape(1, c))
    return out[0, 0]


# speedup vs baseline: 1.7239x; 1.7239x over previous
import functools

import jax
import jax.numpy as jnp
from jax import lax
from jax.experimental import pallas as pl

_S = 30.0


def _half_loss(xb, tgt, ml):
    blk, c = xb.shape
    col = lax.broadcasted_iota(jnp.int32, (blk, c), 1)
    onehot = col == tgt
    m_row = jnp.sum(jnp.where(onehot, ml, 0.0), axis=1, keepdims=True)
    logits = _S * jnp.where(onehot, xb - m_row, xb)
    mx = jnp.max(logits, axis=1, keepdims=True)
    se = jnp.sum(jnp.exp(logits - mx), axis=1, keepdims=True)
    logz = jnp.log(se) + mx
    tgt_logit = jnp.sum(jnp.where(onehot, logits, 0.0), axis=1, keepdims=True)
    return jnp.sum(logz - tgt_logit)


def _ldam_body(x1_ref, x2_ref, tgt1_ref, tgt2_ref, ml_ref, out_ref, *,
               nrows_total):
    ml = ml_ref[...]
    p1 = _half_loss(x1_ref[...], tgt1_ref[...].reshape(-1, 1), ml)
    p2 = _half_loss(x2_ref[...], tgt2_ref[...].reshape(-1, 1), ml)
    part = ((p1 + p2) * (1.0 / nrows_total)).reshape(1, 1)

    @pl.when(pl.program_id(0) == 0)
    def _():
        out_ref[...] = jnp.zeros_like(out_ref)

    out_ref[...] += part


def kernel(x, m_list, target):
    b, c = x.shape
    blk = 2048
    grid = b // blk // 2
    tgt3 = target.reshape(2 * grid, 1, blk)
    out = pl.pallas_call(
        functools.partial(_ldam_body, nrows_total=b),
        grid=(grid,),
        in_specs=[
            pl.BlockSpec((blk, c), lambda i: (i, 0)),
            pl.BlockSpec((blk, c), lambda i: (i + grid, 0)),
            pl.BlockSpec((1, 1, blk), lambda i: (i, 0, 0)),
            pl.BlockSpec((1, 1, blk), lambda i: (i + grid, 0, 0)),
            pl.BlockSpec((1, c), lambda i: (0, 0)),
        ],
        out_specs=pl.BlockSpec((1, 1), lambda i: (0, 0)),
        out_shape=jax.ShapeDtypeStruct((1, 1), jnp.float32),
    )(x, x, tgt3, tgt3, m_list.reshape(1, c))
    return out[0, 0]
